# baseline (speedup 1.0000x reference)
import functools

import jax
import jax.numpy as jnp
from jax import lax
from jax.experimental import pallas as pl
from jax.experimental.pallas import tpu as pltpu
from jax.experimental.pallas import tpu_sc as plsc

_EMBED_DIM = 128
_RULES_SIZE = 1000
_ROWS_PER_TILE = 40
_NUM_ACTIVE = _RULES_SIZE // _ROWS_PER_TILE
_LANES = 16

_mesh = plsc.VectorSubcoreMesh(core_axis_name="c", subcore_axis_name="s")


@functools.partial(
    pl.kernel,
    out_type=jax.ShapeDtypeStruct((_RULES_SIZE,), jnp.float32),
    mesh=_mesh,
    scratch_types=[
        pltpu.VMEM((1,), jnp.int32),
        pltpu.VMEM((1, _EMBED_DIM), jnp.float32),
        pltpu.VMEM((_ROWS_PER_TILE, _EMBED_DIM), jnp.float32),
        pltpu.VMEM((48,), jnp.float32),
        pltpu.VMEM((48,), jnp.float32),
        pltpu.SemaphoreType.DMA,
    ],
    compiler_params=pltpu.CompilerParams(needs_layout_passes=False),
)
def _sc_kernel(ident_hbm, table_hbm, wt_hbm, b_hbm, out_hbm,
               idx_v, row_v, wt_v, b_v, out_v, sem):
    wid = lax.axis_index("s") * 2 + lax.axis_index("c")

    @pl.when(wid < _NUM_ACTIVE)
    def _():
        base = wid * _ROWS_PER_TILE
        pltpu.sync_copy(ident_hbm, idx_v)
        gather = pltpu.async_copy(table_hbm.at[idx_v], row_v, sem)
        pltpu.sync_copy(wt_hbm.at[pl.ds(base, _ROWS_PER_TILE), :], wt_v)
        pltpu.sync_copy(b_hbm.at[pl.ds(base, _ROWS_PER_TILE)],
                        b_v.at[pl.ds(0, _ROWS_PER_TILE)])
        gather.wait()

        row = [row_v[0, pl.ds(_LANES * k, _LANES)]
               for k in range(_EMBED_DIM // _LANES)]
        lane = lax.iota(jnp.int32, _LANES)
        for c in range(_ROWS_PER_TILE // _LANES + 1):
            jmax = min(_LANES, _ROWS_PER_TILE - c * _LANES)
            if jmax <= 0:
                break
            acc = jnp.zeros((_LANES,), jnp.float32)
            for jj in range(jmax):
                j = c * _LANES + jj
                prod = row[0] * wt_v[j, pl.ds(0, _LANES)]
                for k in range(1, _EMBED_DIM // _LANES):
                    prod = prod + row[k] * wt_v[j, pl.ds(_LANES * k, _LANES)]
                sj = lax.reduce_sum_p.bind(prod, axes=(0,))
                acc = jnp.where(lane == jj, sj, acc)
            res = jnp.maximum(acc + b_v[pl.ds(c * _LANES, _LANES)], 0.0)
            out_v[pl.ds(c * _LANES, _LANES)] = res
        pltpu.sync_copy(out_v.at[pl.ds(0, _ROWS_PER_TILE)],
                        out_hbm.at[pl.ds(base, _ROWS_PER_TILE)])


def kernel(ident, table, W, b):
    ident = ident.astype(jnp.int32)
    return _sc_kernel(ident, table, W.T, b).reshape(1, _RULES_SIZE)


# speedup vs baseline: 7.0764x; 7.0764x over previous
import jax
import jax.numpy as jnp
from jax.experimental import pallas as pl
from jax.experimental.pallas import tpu as pltpu

_EMBED_DIM = 128
_RULES_SIZE = 1000
_ROWS_PER_BLOCK = 8


def _fused_kernel(ident_ref, row_ref, wt_hbm, b_ref, out_ref, wt_vmem, sem_w):
    cw = pltpu.make_async_copy(wt_hbm, wt_vmem, sem_w)
    cw.start()
    r = ident_ref[0] % _ROWS_PER_BLOCK
    row = row_ref[pl.ds(r, 1), :]
    cw.wait()
    acc = jax.lax.dot_general(
        row, wt_vmem[...],
        dimension_numbers=(((1,), (1,)), ((), ())),
        preferred_element_type=jnp.float32,
    )
    out_ref[...] = jnp.maximum(acc + b_ref[...], 0.0)


def kernel(ident, table, W, b):
    ident = ident.astype(jnp.int32)
    wt = pltpu.with_memory_space_constraint(W.T, pltpu.MemorySpace.HBM)
    grid_spec = pltpu.PrefetchScalarGridSpec(
        num_scalar_prefetch=1,
        grid=(1,),
        in_specs=[
            pl.BlockSpec(
                (_ROWS_PER_BLOCK, _EMBED_DIM),
                lambda i, ident_ref: (ident_ref[0] // _ROWS_PER_BLOCK, 0),
            ),
            pl.BlockSpec(memory_space=pltpu.MemorySpace.HBM),
            pl.BlockSpec((_RULES_SIZE,), lambda i, ident_ref: (0,)),
        ],
        out_specs=pl.BlockSpec((1, _RULES_SIZE), lambda i, ident_ref: (0, 0)),
        scratch_shapes=[
            pltpu.VMEM((_RULES_SIZE, _EMBED_DIM), jnp.float32),
            pltpu.SemaphoreType.DMA,
        ],
    )
    return pl.pallas_call(
        _fused_kernel,
        grid_spec=grid_spec,
        out_shape=jax.ShapeDtypeStruct((1, _RULES_SIZE), jnp.float32),
        compiler_params=pltpu.CompilerParams(
            disable_bounds_checks=True,
            disable_semaphore_checks=True,
            skip_device_barrier=True,
        ),
    )(ident, table, wt, b)
